# baseline (speedup 1.0000x reference)
import functools

import jax
import jax.numpy as jnp
from jax import lax
from jax.experimental import pallas as pl
from jax.experimental.pallas import tpu as pltpu
from jax.experimental.pallas import tpu_sc as plsc

N = 10000
D = 64
E = 320000
NC, NS = 2, 16
NW = NC * NS
C = 128
NCHUNK = 78
CT = 16
RPT = 624
REM = N - NS * RPT
DEG_W = 16
NBUF = 6

_MESH = plsc.VectorSubcoreMesh(
    core_axis_name="c", subcore_axis_name="s", num_cores=NC, num_subcores=NS)


@functools.partial(
    pl.kernel,
    out_type=jax.ShapeDtypeStruct((NC, N, DEG_W), jnp.float32),
    mesh=_MESH,
    scratch_types=[
        pltpu.VMEM((NCHUNK, C), jnp.int32),
        pltpu.VMEM((CT,), jnp.int32),
        pltpu.VMEM((C, DEG_W), jnp.float32),
        pltpu.VMEM_SHARED((N, DEG_W), jnp.float32),
        pltpu.SemaphoreType.DMA,
    ],
    compiler_params=pltpu.CompilerParams(use_tc_tiling_on_sc=False),
)
def _sc_degree(dstm_hbm, dstt_hbm, ones_hbm, zeros_hbm, out_hbm,
               dst_v, dstt_v, ones_v, acc, sem):
    if True:
        c = lax.axis_index("c")
        s = lax.axis_index("s")
        wid = c * NS + s
        r0 = s * RPT
        pltpu.sync_copy(dstm_hbm.at[pl.ds(wid * NCHUNK, NCHUNK)], dst_v)
        pltpu.sync_copy(dstt_hbm.at[pl.ds(wid * CT, CT)], dstt_v)
        pltpu.sync_copy(ones_hbm, ones_v)
        pltpu.sync_copy(zeros_hbm.at[pl.ds(r0, RPT)], acc.at[pl.ds(r0, RPT)])

        @pl.when(s == NS - 1)
        def _():
            pltpu.sync_copy(zeros_hbm.at[pl.ds(NS * RPT, REM)],
                            acc.at[pl.ds(NS * RPT, REM)])

        plsc.subcore_barrier()

        def chunk(i, carry):
            pltpu.async_copy(ones_v, acc.at[dst_v.at[i]], sem, add=True)
            return carry

        lax.fori_loop(0, NCHUNK, chunk, 0)

        def drain(i, carry):
            pltpu.make_async_copy(ones_v, acc.at[dst_v.at[i]], sem).wait()
            return carry

        lax.fori_loop(0, NCHUNK, drain, 0)
        pltpu.sync_copy(ones_v.at[pl.ds(0, CT)], acc.at[dstt_v], add=True)
        plsc.subcore_barrier()
        pltpu.sync_copy(acc.at[pl.ds(r0, RPT)], out_hbm.at[c, pl.ds(r0, RPT)])

        @pl.when(s == NS - 1)
        def _():
            pltpu.sync_copy(acc.at[pl.ds(NS * RPT, REM)],
                            out_hbm.at[c, pl.ds(NS * RPT, REM)])


@functools.partial(
    pl.kernel,
    out_type=jax.ShapeDtypeStruct((NC, N, D), jnp.float32),
    mesh=_MESH,
    scratch_types=[
        pltpu.VMEM((NCHUNK, C), jnp.int32),
        pltpu.VMEM((NCHUNK, C), jnp.int32),
        pltpu.VMEM((CT,), jnp.int32),
        pltpu.VMEM((CT,), jnp.int32),
        pltpu.VMEM((NBUF, C, D), jnp.float32),
        pltpu.VMEM_SHARED((N, D), jnp.float32),
    ] + [pltpu.SemaphoreType.DMA] * (2 * NBUF),
    compiler_params=pltpu.CompilerParams(use_tc_tiling_on_sc=False),
)
def _sc_aggregate(hn_hbm, srcm_hbm, dstm_hbm, srct_hbm, dstt_hbm, zeros_hbm,
                  out_hbm, src_v, dst_v, srct_v, dstt_v, rows_v, acc, *sems):
    if True:
        sg, ss = sems[:NBUF], sems[NBUF:]
        c = lax.axis_index("c")
        s = lax.axis_index("s")
        wid = c * NS + s
        r0 = s * RPT
        pltpu.sync_copy(srcm_hbm.at[pl.ds(wid * NCHUNK, NCHUNK)], src_v)
        pltpu.sync_copy(dstm_hbm.at[pl.ds(wid * NCHUNK, NCHUNK)], dst_v)
        pltpu.sync_copy(srct_hbm.at[pl.ds(wid * CT, CT)], srct_v)
        pltpu.sync_copy(dstt_hbm.at[pl.ds(wid * CT, CT)], dstt_v)
        pltpu.sync_copy(zeros_hbm.at[pl.ds(r0, RPT)], acc.at[pl.ds(r0, RPT)])

        @pl.when(s == NS - 1)
        def _():
            pltpu.sync_copy(zeros_hbm.at[pl.ds(NS * RPT, REM)],
                            acc.at[pl.ds(NS * RPT, REM)])

        plsc.subcore_barrier()

        def gather(i, b):
            return pltpu.async_copy(
                hn_hbm.at[src_v.at[i]], rows_v.at[b], sg[b])

        def scatter(i, b):
            return pltpu.async_copy(
                rows_v.at[b], acc.at[dst_v.at[i]], ss[b], add=True)

        for b in range(NBUF):
            gather(b, b)

        def outer(o, carry):
            i0 = o * NBUF
            for b in range(NBUF):
                pltpu.make_async_copy(
                    hn_hbm.at[src_v.at[i0 + b]], rows_v.at[b], sg[b]).wait()
                scatter(i0 + b, b)
            for b in range(NBUF):
                pltpu.make_async_copy(
                    rows_v.at[b], acc.at[dst_v.at[i0 + b]], ss[b]).wait()

                @pl.when(i0 + b + NBUF < NCHUNK)
                def _():
                    gather(i0 + b + NBUF, b)
            return carry

        lax.fori_loop(0, NCHUNK // NBUF, outer, 0)
        pltpu.sync_copy(hn_hbm.at[srct_v], rows_v.at[0, pl.ds(0, CT)])
        pltpu.sync_copy(rows_v.at[0, pl.ds(0, CT)], acc.at[dstt_v], add=True)
        plsc.subcore_barrier()
        pltpu.sync_copy(acc.at[pl.ds(r0, RPT)], out_hbm.at[c, pl.ds(r0, RPT)])

        @pl.when(s == NS - 1)
        def _():
            pltpu.sync_copy(acc.at[pl.ds(NS * RPT, REM)],
                            out_hbm.at[c, pl.ds(NS * RPT, REM)])


def _tc_mm(x_ref, w_ref, h0_ref):
    h0_ref[...] = jnp.dot(
        x_ref[...], w_ref[...], preferred_element_type=jnp.float32)


def _tc_pre(h0_ref, degp_ref, hn0_ref, dinv_ref):
    deg = degp_ref[0, :, 0:1] + degp_ref[1, :, 0:1] + 1.0
    dinv = lax.rsqrt(deg)
    hn0_ref[:, 0:D] = h0_ref[...] * dinv
    dinv_ref[...] = dinv


_GRID = 10
BR = N // _GRID


def _tc_mid(accp_ref, hn0_ref, dinv_ref, bg1_ref, wg2_ref, hn1_ref):
    agg = accp_ref[0] + accp_ref[1] + hn0_ref[:, 0:D]
    dinv = dinv_ref[...]
    h1 = jnp.maximum(dinv * agg + bg1_ref[...], 0.0)
    hn1_ref[:, 0:D] = jnp.dot(
        h1, wg2_ref[...], preferred_element_type=jnp.float32) * dinv


def _tc_head(accp_ref, hn1_ref, dinv_ref, bg2_ref, wf1_ref, bf1_ref,
             wf2_ref, bf2_ref, wf3_ref, bf3_ref, out_ref, psum_ref):
    i = pl.program_id(0)
    agg = accp_ref[0] + accp_ref[1] + hn1_ref[:, 0:D]
    h2 = jnp.maximum(dinv_ref[...] * agg + bg2_ref[...], 0.0)
    blk = jnp.sum(h2, axis=0, keepdims=True)

    @pl.when(i == 0)
    def _():
        psum_ref[...] = blk

    @pl.when(i > 0)
    def _():
        psum_ref[...] += blk

    @pl.when(i == _GRID - 1)
    def _():
        g = jnp.maximum(psum_ref[...] * (1.0 / N), 0.0)
        g = jnp.maximum(
            jnp.dot(g, wf1_ref[...], preferred_element_type=jnp.float32)
            + bf1_ref[...], 0.0)
        g = jnp.maximum(
            jnp.dot(g, wf2_ref[...], preferred_element_type=jnp.float32)
            + bf2_ref[...], 0.0)
        out_ref[...] = jnp.dot(
            g, wf3_ref[...], preferred_element_type=jnp.float32) + bf3_ref[...]


def kernel(x, edge_index, Wg1, bg1, Wg2, bg2, Wf1, bf1, Wf2, bf2, Wf3, bf3):
    src = edge_index[0].astype(jnp.int32)
    dst = edge_index[1].astype(jnp.int32)
    nm = NW * NCHUNK * C
    src2m = (src[:nm] * 2).reshape(NW * NCHUNK, C)
    dstm = dst[:nm].reshape(NW * NCHUNK, C)
    src2t = src[nm:] * 2
    dstt = dst[nm:]
    zeros_d = jnp.zeros((N, D), jnp.float32)
    zeros_deg = jnp.zeros((N, DEG_W), jnp.float32)
    ones_deg = jnp.ones((C, DEG_W), jnp.float32)

    degp = _sc_degree(dstm, dstt, ones_deg, zeros_deg)

    h0 = pl.pallas_call(
        _tc_mm,
        out_shape=jax.ShapeDtypeStruct((N, D), jnp.float32),
    )(x, Wg1)

    hn0, dinv = pl.pallas_call(
        _tc_pre,
        grid=(_GRID,),
        in_specs=[pl.BlockSpec((BR, D), lambda i: (i, 0)),
                  pl.BlockSpec((NC, BR, DEG_W), lambda i: (0, i, 0))],
        out_specs=(pl.BlockSpec((BR, 2 * D), lambda i: (i, 0)),
                   pl.BlockSpec((BR, 1), lambda i: (i, 0))),
        out_shape=(jax.ShapeDtypeStruct((N, 2 * D), jnp.float32),
                   jax.ShapeDtypeStruct((N, 1), jnp.float32)),
    )(h0, degp)

    acc1 = _sc_aggregate(hn0.reshape(2 * N, D), src2m, dstm, src2t, dstt,
                         zeros_d)

    hn1 = pl.pallas_call(
        _tc_mid,
        grid=(_GRID,),
        in_specs=[pl.BlockSpec((NC, BR, D), lambda i: (0, i, 0)),
                  pl.BlockSpec((BR, 2 * D), lambda i: (i, 0)),
                  pl.BlockSpec((BR, 1), lambda i: (i, 0)),
                  pl.BlockSpec((1, D), lambda i: (0, 0)),
                  pl.BlockSpec((D, D), lambda i: (0, 0))],
        out_specs=pl.BlockSpec((BR, 2 * D), lambda i: (i, 0)),
        out_shape=jax.ShapeDtypeStruct((N, 2 * D), jnp.float32),
    )(acc1, hn0, dinv, bg1.reshape(1, -1), Wg2)

    acc2 = _sc_aggregate(hn1.reshape(2 * N, D), src2m, dstm, src2t, dstt,
                         zeros_d)

    full = pl.BlockSpec(None, lambda i: tuple(0 for _ in range(2)))
    out = pl.pallas_call(
        _tc_head,
        grid=(_GRID,),
        in_specs=[pl.BlockSpec((NC, BR, D), lambda i: (0, i, 0)),
                  pl.BlockSpec((BR, 2 * D), lambda i: (i, 0)),
                  pl.BlockSpec((BR, 1), lambda i: (i, 0)),
                  pl.BlockSpec((1, D), lambda i: (0, 0)),
                  pl.BlockSpec((D, D), lambda i: (0, 0)),
                  pl.BlockSpec((1, D), lambda i: (0, 0)),
                  pl.BlockSpec((D, D), lambda i: (0, 0)),
                  pl.BlockSpec((1, D), lambda i: (0, 0)),
                  pl.BlockSpec((D, 32), lambda i: (0, 0)),
                  pl.BlockSpec((1, 32), lambda i: (0, 0))],
        out_specs=pl.BlockSpec((1, 32), lambda i: (0, 0)),
        out_shape=jax.ShapeDtypeStruct((1, 32), jnp.float32),
        scratch_shapes=[pltpu.VMEM((1, D), jnp.float32)],
    )(acc2, hn1, dinv, bg2.reshape(1, -1), Wf1, bf1.reshape(1, -1),
      Wf2, bf2.reshape(1, -1), Wf3, bf3.reshape(1, -1))
    return out


# speedup vs baseline: 1.0107x; 1.0107x over previous
import functools

import jax
import jax.numpy as jnp
from jax import lax
from jax.experimental import pallas as pl
from jax.experimental.pallas import tpu as pltpu
from jax.experimental.pallas import tpu_sc as plsc

N = 10000
D = 64
E = 320000
NC, NS = 2, 16
NW = NC * NS
C = 128
NCHUNK = 78
CT = 16
RPT = 624
REM = N - NS * RPT
DEG_W = 16
NBUF = 6

_MESH = plsc.VectorSubcoreMesh(
    core_axis_name="c", subcore_axis_name="s", num_cores=NC, num_subcores=NS)


@functools.partial(
    pl.kernel,
    out_type=jax.ShapeDtypeStruct((NC, N, DEG_W), jnp.float32),
    mesh=_MESH,
    scratch_types=[
        pltpu.VMEM((NCHUNK, C), jnp.int32),
        pltpu.VMEM((CT,), jnp.int32),
        pltpu.VMEM((C, DEG_W), jnp.float32),
        pltpu.VMEM_SHARED((N, DEG_W), jnp.float32),
        pltpu.SemaphoreType.DMA,
    ],
    compiler_params=pltpu.CompilerParams(use_tc_tiling_on_sc=False),
)
def _sc_degree(dstm_hbm, dstt_hbm, ones_hbm, zeros_hbm, out_hbm,
               dst_v, dstt_v, ones_v, acc, sem):
    if True:
        c = lax.axis_index("c")
        s = lax.axis_index("s")
        wid = c * NS + s
        r0 = s * RPT
        pltpu.sync_copy(dstm_hbm.at[pl.ds(wid * NCHUNK, NCHUNK)], dst_v)
        pltpu.sync_copy(dstt_hbm.at[pl.ds(wid * CT, CT)], dstt_v)
        pltpu.sync_copy(ones_hbm, ones_v)
        pltpu.sync_copy(zeros_hbm.at[pl.ds(r0, RPT)], acc.at[pl.ds(r0, RPT)])

        @pl.when(s == NS - 1)
        def _():
            pltpu.sync_copy(zeros_hbm.at[pl.ds(NS * RPT, REM)],
                            acc.at[pl.ds(NS * RPT, REM)])

        plsc.subcore_barrier()

        def chunk(i, carry):
            pltpu.async_copy(ones_v, acc.at[dst_v.at[i]], sem, add=True)
            return carry

        lax.fori_loop(0, NCHUNK, chunk, 0)

        def drain(i, carry):
            pltpu.make_async_copy(ones_v, acc.at[dst_v.at[i]], sem).wait()
            return carry

        lax.fori_loop(0, NCHUNK, drain, 0)
        pltpu.sync_copy(ones_v.at[pl.ds(0, CT)], acc.at[dstt_v], add=True)
        plsc.subcore_barrier()
        pltpu.sync_copy(acc.at[pl.ds(r0, RPT)], out_hbm.at[c, pl.ds(r0, RPT)])

        @pl.when(s == NS - 1)
        def _():
            pltpu.sync_copy(acc.at[pl.ds(NS * RPT, REM)],
                            out_hbm.at[c, pl.ds(NS * RPT, REM)])


@functools.partial(
    pl.kernel,
    out_type=jax.ShapeDtypeStruct((NC, N, D), jnp.float32),
    mesh=_MESH,
    scratch_types=[
        pltpu.VMEM((NCHUNK, C), jnp.int32),
        pltpu.VMEM((NCHUNK, C), jnp.int32),
        pltpu.VMEM((CT,), jnp.int32),
        pltpu.VMEM((CT,), jnp.int32),
        pltpu.VMEM((NBUF, C, D), jnp.float32),
        pltpu.VMEM_SHARED((N, D), jnp.float32),
    ] + [pltpu.SemaphoreType.DMA] * (2 * NBUF),
    compiler_params=pltpu.CompilerParams(use_tc_tiling_on_sc=False),
)
def _sc_aggregate(hn_hbm, srcm_hbm, dstm_hbm, srct_hbm, dstt_hbm, zeros_hbm,
                  out_hbm, src_v, dst_v, srct_v, dstt_v, rows_v, acc, *sems):
    if True:
        sg, ss = sems[:NBUF], sems[NBUF:]
        c = lax.axis_index("c")
        s = lax.axis_index("s")
        wid = c * NS + s
        r0 = s * RPT
        pltpu.sync_copy(srcm_hbm.at[pl.ds(wid * NCHUNK, NCHUNK)], src_v)
        pltpu.sync_copy(dstm_hbm.at[pl.ds(wid * NCHUNK, NCHUNK)], dst_v)
        pltpu.sync_copy(srct_hbm.at[pl.ds(wid * CT, CT)], srct_v)
        pltpu.sync_copy(dstt_hbm.at[pl.ds(wid * CT, CT)], dstt_v)
        pltpu.sync_copy(zeros_hbm.at[pl.ds(r0, RPT)], acc.at[pl.ds(r0, RPT)])

        @pl.when(s == NS - 1)
        def _():
            pltpu.sync_copy(zeros_hbm.at[pl.ds(NS * RPT, REM)],
                            acc.at[pl.ds(NS * RPT, REM)])

        plsc.subcore_barrier()

        def gather(i, b):
            return pltpu.async_copy(
                hn_hbm.at[src_v.at[i]], rows_v.at[b], sg[b])

        def scatter(i, b):
            return pltpu.async_copy(
                rows_v.at[b], acc.at[dst_v.at[i]], ss[b], add=True)

        for b in range(NBUF):
            gather(b, b)

        def outer(o, carry):
            i0 = o * NBUF
            for b in range(NBUF):
                pltpu.make_async_copy(
                    hn_hbm.at[src_v.at[i0 + b]], rows_v.at[b], sg[b]).wait()
                scatter(i0 + b, b)
            for b in range(NBUF):
                pltpu.make_async_copy(
                    rows_v.at[b], acc.at[dst_v.at[i0 + b]], ss[b]).wait()

                @pl.when(i0 + b + NBUF < NCHUNK)
                def _():
                    gather(i0 + b + NBUF, b)
            return carry

        lax.fori_loop(0, NCHUNK // NBUF, outer, 0)
        pltpu.sync_copy(hn_hbm.at[srct_v], rows_v.at[0, pl.ds(0, CT)])
        pltpu.sync_copy(rows_v.at[0, pl.ds(0, CT)], acc.at[dstt_v], add=True)
        plsc.subcore_barrier()
        pltpu.sync_copy(acc.at[pl.ds(r0, RPT)], out_hbm.at[c, pl.ds(r0, RPT)])

        @pl.when(s == NS - 1)
        def _():
            pltpu.sync_copy(acc.at[pl.ds(NS * RPT, REM)],
                            out_hbm.at[c, pl.ds(NS * RPT, REM)])


def _tc_mm(x_ref, w_ref, h0_ref):
    h0_ref[...] = jnp.dot(
        x_ref[...], w_ref[...], preferred_element_type=jnp.float32)


def _tc_pre(h0_ref, degp_ref, hn0_ref, dinv_ref):
    deg = degp_ref[0, :, 0:1] + degp_ref[1, :, 0:1] + 1.0
    dinv = lax.rsqrt(deg)
    hn0_ref[:, 0:D] = h0_ref[...] * dinv
    dinv_ref[...] = dinv


_GRID = 10
BR = N // _GRID


def _tc_mid(accp_ref, hn0_ref, dinv_ref, bg1_ref, wg2_ref, hn1_ref):
    agg = accp_ref[0] + accp_ref[1] + hn0_ref[:, 0:D]
    dinv = dinv_ref[...]
    h1 = jnp.maximum(dinv * agg + bg1_ref[...], 0.0)
    hn1_ref[:, 0:D] = jnp.dot(
        h1, wg2_ref[...], preferred_element_type=jnp.float32) * dinv


def _tc_head(accp_ref, hn1_ref, dinv_ref, bg2_ref, wf1_ref, bf1_ref,
             wf2_ref, bf2_ref, wf3_ref, bf3_ref, out_ref):
    agg = accp_ref[0] + accp_ref[1] + hn1_ref[:, 0:D]
    h2 = jnp.maximum(dinv_ref[...] * agg + bg2_ref[...], 0.0)
    g = jnp.maximum(jnp.mean(h2, axis=0, keepdims=True), 0.0)
    g = jnp.maximum(
        jnp.dot(g, wf1_ref[...], preferred_element_type=jnp.float32)
        + bf1_ref[...], 0.0)
    g = jnp.maximum(
        jnp.dot(g, wf2_ref[...], preferred_element_type=jnp.float32)
        + bf2_ref[...], 0.0)
    out_ref[...] = jnp.dot(
        g, wf3_ref[...], preferred_element_type=jnp.float32) + bf3_ref[...]


def kernel(x, edge_index, Wg1, bg1, Wg2, bg2, Wf1, bf1, Wf2, bf2, Wf3, bf3):
    src = edge_index[0].astype(jnp.int32)
    dst = edge_index[1].astype(jnp.int32)
    nm = NW * NCHUNK * C
    src2m = (src[:nm] * 2).reshape(NW * NCHUNK, C)
    dstm = dst[:nm].reshape(NW * NCHUNK, C)
    src2t = src[nm:] * 2
    dstt = dst[nm:]
    zeros_d = jnp.zeros((N, D), jnp.float32)
    zeros_deg = jnp.zeros((N, DEG_W), jnp.float32)
    ones_deg = jnp.ones((C, DEG_W), jnp.float32)

    degp = _sc_degree(dstm, dstt, ones_deg, zeros_deg)

    h0 = pl.pallas_call(
        _tc_mm,
        out_shape=jax.ShapeDtypeStruct((N, D), jnp.float32),
    )(x, Wg1)

    hn0, dinv = pl.pallas_call(
        _tc_pre,
        grid=(_GRID,),
        in_specs=[pl.BlockSpec((BR, D), lambda i: (i, 0)),
                  pl.BlockSpec((NC, BR, DEG_W), lambda i: (0, i, 0))],
        out_specs=(pl.BlockSpec((BR, 2 * D), lambda i: (i, 0)),
                   pl.BlockSpec((BR, 1), lambda i: (i, 0))),
        out_shape=(jax.ShapeDtypeStruct((N, 2 * D), jnp.float32),
                   jax.ShapeDtypeStruct((N, 1), jnp.float32)),
    )(h0, degp)

    acc1 = _sc_aggregate(hn0.reshape(2 * N, D), src2m, dstm, src2t, dstt,
                         zeros_d)

    hn1 = pl.pallas_call(
        _tc_mid,
        grid=(_GRID,),
        in_specs=[pl.BlockSpec((NC, BR, D), lambda i: (0, i, 0)),
                  pl.BlockSpec((BR, 2 * D), lambda i: (i, 0)),
                  pl.BlockSpec((BR, 1), lambda i: (i, 0)),
                  pl.BlockSpec((1, D), lambda i: (0, 0)),
                  pl.BlockSpec((D, D), lambda i: (0, 0))],
        out_specs=pl.BlockSpec((BR, 2 * D), lambda i: (i, 0)),
        out_shape=jax.ShapeDtypeStruct((N, 2 * D), jnp.float32),
    )(acc1, hn0, dinv, bg1.reshape(1, -1), Wg2)

    acc2 = _sc_aggregate(hn1.reshape(2 * N, D), src2m, dstm, src2t, dstt,
                         zeros_d)

    out = pl.pallas_call(
        _tc_head,
        out_shape=jax.ShapeDtypeStruct((1, 32), jnp.float32),
    )(acc2, hn1, dinv, bg2.reshape(1, -1), Wf1, bf1.reshape(1, -1),
      Wf2, bf2.reshape(1, -1), Wf3, bf3.reshape(1, -1))
    return out


# speedup vs baseline: 1.0292x; 1.0183x over previous
import functools

import jax
import jax.numpy as jnp
from jax import lax
from jax.experimental import pallas as pl
from jax.experimental.pallas import tpu as pltpu
from jax.experimental.pallas import tpu_sc as plsc

N = 10000
D = 64
E = 320000
NC, NS = 2, 16
NW = NC * NS
C = 128
NCHUNK = 78
CT = 16
RPT = 624
REM = N - NS * RPT
DEG_W = 16
NBUF = 6

_MESH = plsc.VectorSubcoreMesh(
    core_axis_name="c", subcore_axis_name="s", num_cores=NC, num_subcores=NS)


@functools.partial(
    pl.kernel,
    out_type=jax.ShapeDtypeStruct((NC, N, DEG_W), jnp.float32),
    mesh=_MESH,
    scratch_types=[
        pltpu.VMEM((NCHUNK, C), jnp.int32),
        pltpu.VMEM((CT,), jnp.int32),
        pltpu.VMEM((C, DEG_W), jnp.float32),
        pltpu.VMEM_SHARED((N, DEG_W), jnp.float32),
        pltpu.SemaphoreType.DMA,
    ],
    compiler_params=pltpu.CompilerParams(use_tc_tiling_on_sc=False),
)
def _sc_degree(dstm_hbm, dstt_hbm, ones_hbm, zeros_hbm, out_hbm,
               dst_v, dstt_v, ones_v, acc, sem):
    if True:
        c = lax.axis_index("c")
        s = lax.axis_index("s")
        wid = c * NS + s
        r0 = s * RPT
        pltpu.sync_copy(dstm_hbm.at[pl.ds(wid * NCHUNK, NCHUNK)], dst_v)
        pltpu.sync_copy(dstt_hbm.at[pl.ds(wid * CT, CT)], dstt_v)
        pltpu.sync_copy(ones_hbm, ones_v)
        pltpu.sync_copy(zeros_hbm.at[pl.ds(r0, RPT)], acc.at[pl.ds(r0, RPT)])

        @pl.when(s == NS - 1)
        def _():
            pltpu.sync_copy(zeros_hbm.at[pl.ds(NS * RPT, REM)],
                            acc.at[pl.ds(NS * RPT, REM)])

        plsc.subcore_barrier()

        def chunk(i, carry):
            pltpu.async_copy(ones_v, acc.at[dst_v.at[i]], sem, add=True)
            return carry

        lax.fori_loop(0, NCHUNK, chunk, 0)

        def drain(i, carry):
            pltpu.make_async_copy(ones_v, acc.at[dst_v.at[i]], sem).wait()
            return carry

        lax.fori_loop(0, NCHUNK, drain, 0)
        pltpu.sync_copy(ones_v.at[pl.ds(0, CT)], acc.at[dstt_v], add=True)
        plsc.subcore_barrier()
        pltpu.sync_copy(acc.at[pl.ds(r0, RPT)], out_hbm.at[c, pl.ds(r0, RPT)])

        @pl.when(s == NS - 1)
        def _():
            pltpu.sync_copy(acc.at[pl.ds(NS * RPT, REM)],
                            out_hbm.at[c, pl.ds(NS * RPT, REM)])


@functools.partial(
    pl.kernel,
    out_type=jax.ShapeDtypeStruct((NC, N, D), jnp.float32),
    mesh=_MESH,
    scratch_types=[
        pltpu.VMEM((NCHUNK, C), jnp.int32),
        pltpu.VMEM((NCHUNK, C), jnp.int32),
        pltpu.VMEM((CT,), jnp.int32),
        pltpu.VMEM((CT,), jnp.int32),
        pltpu.VMEM((NBUF, C, D), jnp.float32),
        pltpu.VMEM_SHARED((N, D), jnp.float32),
    ] + [pltpu.SemaphoreType.DMA] * (2 * NBUF),
    compiler_params=pltpu.CompilerParams(use_tc_tiling_on_sc=False),
)
def _sc_aggregate(hn_hbm, srcm_hbm, dstm_hbm, srct_hbm, dstt_hbm, zeros_hbm,
                  out_hbm, src_v, dst_v, srct_v, dstt_v, rows_v, acc, *sems):
    if True:
        sg, ss = sems[:NBUF], sems[NBUF:]
        c = lax.axis_index("c")
        s = lax.axis_index("s")
        wid = c * NS + s
        r0 = s * RPT
        pltpu.sync_copy(srcm_hbm.at[pl.ds(wid * NCHUNK, NCHUNK)], src_v)
        pltpu.sync_copy(dstm_hbm.at[pl.ds(wid * NCHUNK, NCHUNK)], dst_v)
        pltpu.sync_copy(srct_hbm.at[pl.ds(wid * CT, CT)], srct_v)
        pltpu.sync_copy(dstt_hbm.at[pl.ds(wid * CT, CT)], dstt_v)
        pltpu.sync_copy(zeros_hbm.at[pl.ds(r0, RPT)], acc.at[pl.ds(r0, RPT)])

        @pl.when(s == NS - 1)
        def _():
            pltpu.sync_copy(zeros_hbm.at[pl.ds(NS * RPT, REM)],
                            acc.at[pl.ds(NS * RPT, REM)])

        plsc.subcore_barrier()

        def gather(i, b):
            return pltpu.async_copy(
                hn_hbm.at[src_v.at[i]], rows_v.at[b], sg[b])

        def scatter(i, b):
            return pltpu.async_copy(
                rows_v.at[b], acc.at[dst_v.at[i]], ss[b], add=True)

        for b in range(NBUF):
            gather(b, b)

        def outer(o, carry):
            i0 = o * NBUF
            for b in range(NBUF):
                pltpu.make_async_copy(
                    hn_hbm.at[src_v.at[i0 + b]], rows_v.at[b], sg[b]).wait()
                scatter(i0 + b, b)
            for b in range(NBUF):
                pltpu.make_async_copy(
                    rows_v.at[b], acc.at[dst_v.at[i0 + b]], ss[b]).wait()

                @pl.when(i0 + b + NBUF < NCHUNK)
                def _():
                    gather(i0 + b + NBUF, b)
            return carry

        lax.fori_loop(0, NCHUNK // NBUF, outer, 0)
        pltpu.sync_copy(hn_hbm.at[srct_v], rows_v.at[0, pl.ds(0, CT)])
        pltpu.sync_copy(rows_v.at[0, pl.ds(0, CT)], acc.at[dstt_v], add=True)
        plsc.subcore_barrier()
        pltpu.sync_copy(acc.at[pl.ds(r0, RPT)], out_hbm.at[c, pl.ds(r0, RPT)])

        @pl.when(s == NS - 1)
        def _():
            pltpu.sync_copy(acc.at[pl.ds(NS * RPT, REM)],
                            out_hbm.at[c, pl.ds(NS * RPT, REM)])


def _tc_mm(x_ref, w_ref, h0_ref):
    h0_ref[...] = jnp.dot(
        x_ref[...], w_ref[...], preferred_element_type=jnp.float32)


def _tc_pre(h0_ref, degp_ref, hn0_ref, dinv_ref):
    deg = degp_ref[0, :, 0:1] + degp_ref[1, :, 0:1] + 1.0
    dinv = lax.rsqrt(deg)
    hn0_ref[:, 0:D] = h0_ref[...] * dinv
    dinv_ref[...] = dinv


def _tc_mid(accp_ref, hn0_ref, dinv_ref, bg1_ref, wg2_ref, hn1_ref):
    agg = accp_ref[0] + accp_ref[1] + hn0_ref[:, 0:D]
    dinv = dinv_ref[...]
    h1 = jnp.maximum(dinv * agg + bg1_ref[...], 0.0)
    hn1_ref[:, 0:D] = jnp.dot(
        h1, wg2_ref[...], preferred_element_type=jnp.float32) * dinv


def _tc_head(accp_ref, hn1_ref, dinv_ref, bg2_ref, wf1_ref, bf1_ref,
             wf2_ref, bf2_ref, wf3_ref, bf3_ref, out_ref):
    agg = accp_ref[0] + accp_ref[1] + hn1_ref[:, 0:D]
    h2 = jnp.maximum(dinv_ref[...] * agg + bg2_ref[...], 0.0)
    g = jnp.maximum(jnp.mean(h2, axis=0, keepdims=True), 0.0)
    g = jnp.maximum(
        jnp.dot(g, wf1_ref[...], preferred_element_type=jnp.float32)
        + bf1_ref[...], 0.0)
    g = jnp.maximum(
        jnp.dot(g, wf2_ref[...], preferred_element_type=jnp.float32)
        + bf2_ref[...], 0.0)
    out_ref[...] = jnp.dot(
        g, wf3_ref[...], preferred_element_type=jnp.float32) + bf3_ref[...]


def kernel(x, edge_index, Wg1, bg1, Wg2, bg2, Wf1, bf1, Wf2, bf2, Wf3, bf3):
    src = edge_index[0].astype(jnp.int32)
    dst = edge_index[1].astype(jnp.int32)
    nm = NW * NCHUNK * C
    src2m = (src[:nm] * 2).reshape(NW * NCHUNK, C)
    dstm = dst[:nm].reshape(NW * NCHUNK, C)
    src2t = src[nm:] * 2
    dstt = dst[nm:]
    zeros_d = jnp.zeros((N, D), jnp.float32)
    zeros_deg = jnp.zeros((N, DEG_W), jnp.float32)
    ones_deg = jnp.ones((C, DEG_W), jnp.float32)

    degp = _sc_degree(dstm, dstt, ones_deg, zeros_deg)

    h0 = pl.pallas_call(
        _tc_mm,
        out_shape=jax.ShapeDtypeStruct((N, D), jnp.float32),
    )(x, Wg1)

    hn0, dinv = pl.pallas_call(
        _tc_pre,
        out_shape=(jax.ShapeDtypeStruct((N, 2 * D), jnp.float32),
                   jax.ShapeDtypeStruct((N, 1), jnp.float32)),
    )(h0, degp)

    acc1 = _sc_aggregate(hn0.reshape(2 * N, D), src2m, dstm, src2t, dstt,
                         zeros_d)

    hn1 = pl.pallas_call(
        _tc_mid,
        out_shape=jax.ShapeDtypeStruct((N, 2 * D), jnp.float32),
    )(acc1, hn0, dinv, bg1.reshape(1, -1), Wg2)

    acc2 = _sc_aggregate(hn1.reshape(2 * N, D), src2m, dstm, src2t, dstt,
                         zeros_d)

    out = pl.pallas_call(
        _tc_head,
        out_shape=jax.ShapeDtypeStruct((1, 32), jnp.float32),
    )(acc2, hn1, dinv, bg2.reshape(1, -1), Wf1, bf1.reshape(1, -1),
      Wf2, bf2.reshape(1, -1), Wf3, bf3.reshape(1, -1))
    return out
